# baseline (speedup 1.0000x reference)
import functools

import jax
import jax.numpy as jnp
import numpy as np
from jax import lax
from jax.experimental import pallas as pl
from jax.experimental.pallas import tpu as pltpu
from jax.experimental.pallas import tpu_sc as plsc

VOCAB = 1000000
D = 64
BATCH = 1024
SEQ = 200
NW = 32
NTB = 8
BB = BATCH // NTB
NSG = NW // NTB
SG = SEQ // NSG
NBUF = 5
SCALE = 8.0


def _pos_encoding() -> np.ndarray:
    depth = D / 2
    positions = np.arange(SEQ)[:, np.newaxis]
    depths = np.arange(depth)[np.newaxis, :] / depth
    angle_rates = 1 / 10000 ** depths
    angle_rads = positions * angle_rates
    pe = np.concatenate([np.sin(angle_rads), np.cos(angle_rads)], axis=-1)
    return np.asarray(pe, dtype=np.float32)


_POS = _pos_encoding()

_mesh = plsc.VectorSubcoreMesh(core_axis_name="c", subcore_axis_name="s")


@functools.partial(
    pl.kernel,
    mesh=_mesh,
    compiler_params=pltpu.CompilerParams(
        use_tc_tiling_on_sc=False, needs_layout_passes=False
    ),
    out_type=jax.ShapeDtypeStruct((SEQ, D // 8, NTB, 8, BB), jnp.float32),
    scratch_types=[
        pltpu.VMEM((SG, BB), jnp.int32),
        pltpu.VMEM((SG, D), jnp.float32),
        pltpu.VMEM((NBUF, BB, D), jnp.float32),
        pltpu.VMEM((NBUF, D // 8, 8, BB), jnp.float32),
        pltpu.SemaphoreType.DMA,
        pltpu.SemaphoreType.DMA,
    ],
)
def _embed_kernel(
    xt_hbm, pos_hbm, table_hbm, out_hbm, idx_v, pos_v, bufs, obufs, gsem, wsem
):
    cc = lax.axis_index("c")
    ss = lax.axis_index("s")
    wid = ss * 2 + cc
    tb = wid % NTB
    s0 = (wid // NTB) * SG
    b0 = tb * BB

    pltpu.sync_copy(xt_hbm.at[pl.ds(s0, SG), pl.ds(b0, BB)], idx_v)
    pltpu.sync_copy(pos_hbm.at[pl.ds(s0, SG)], pos_v)

    for b in range(NBUF):
        pltpu.async_copy(table_hbm.at[idx_v.at[b]], bufs.at[b], gsem)

    iota16 = lax.broadcasted_iota(jnp.int32, (16,), 0)
    rowidx = [iota16 + (k * 16) for k in range(BB // 16)]
    zeros16 = jnp.zeros((16,), jnp.int32)

    def outer(o, carry):
        for b in range(NBUF):
            blk = o * NBUF + b
            buf = bufs.at[b]
            obuf = obufs.at[b]
            pltpu.make_async_copy(table_hbm.at[idx_v.at[blk]], buf, gsem).wait()
            @pl.when(blk >= NBUF)
            def _():
                pltpu.make_async_copy(obuf, out_hbm.at[s0, :, tb], wsem).wait()

            blk16 = zeros16 + blk

            def ecol(e, carry2):
                e16 = zeros16 + e
                q = plsc.load_gather(pos_v, [blk16, e16]) * 1.0
                te = e // 8
                ei = e % 8
                for k in range(BB // 16):
                    g = plsc.load_gather(buf, [rowidx[k], e16])
                    obuf[te, ei, pl.ds(k * 16, 16)] = g * SCALE + q
                return carry2

            lax.fori_loop(0, D, ecol, 0)

            pltpu.async_copy(obuf, out_hbm.at[s0 + blk, :, tb], wsem)

            @pl.when(blk + NBUF < SG)
            def _():
                pltpu.async_copy(table_hbm.at[idx_v.at[blk + NBUF]], bufs.at[b], gsem)
        return carry

    lax.fori_loop(0, SG // NBUF, outer, 0)

    for b in range(NBUF):
        pltpu.make_async_copy(obufs.at[b], out_hbm.at[s0, :, tb], wsem).wait()


def kernel(x, table):
    xt = jnp.transpose(x.astype(jnp.int32))
    out = _embed_kernel(xt, _POS, table)
    return jnp.reshape(jnp.transpose(out, (2, 4, 0, 1, 3)), (BATCH, SEQ, D))


# speedup vs baseline: 1.0569x; 1.0569x over previous
import functools

import jax
import jax.numpy as jnp
import numpy as np
from jax import lax
from jax.experimental import pallas as pl
from jax.experimental.pallas import tpu as pltpu
from jax.experimental.pallas import tpu_sc as plsc

VOCAB = 1000000
D = 64
BATCH = 1024
SEQ = 200
NW = 32
NTB = 8
BB = BATCH // NTB
NSG = NW // NTB
SG = SEQ // NSG
NBUF = 5
SCALE = 8.0


def _pos_encoding() -> np.ndarray:
    depth = D / 2
    positions = np.arange(SEQ)[:, np.newaxis]
    depths = np.arange(depth)[np.newaxis, :] / depth
    angle_rates = 1 / 10000 ** depths
    angle_rads = positions * angle_rates
    pe = np.concatenate([np.sin(angle_rads), np.cos(angle_rads)], axis=-1)
    return np.asarray(pe, dtype=np.float32)


_POS = _pos_encoding()

_mesh = plsc.VectorSubcoreMesh(core_axis_name="c", subcore_axis_name="s")


@functools.partial(
    pl.kernel,
    mesh=_mesh,
    compiler_params=pltpu.CompilerParams(
        use_tc_tiling_on_sc=False, needs_layout_passes=False
    ),
    out_type=jax.ShapeDtypeStruct((SEQ, D // 8, NTB, 8, BB), jnp.float32),
    scratch_types=[
        pltpu.VMEM((SG, BB), jnp.int32),
        pltpu.VMEM((SG, D), jnp.float32),
        pltpu.VMEM((NBUF, BB, D), jnp.float32),
        pltpu.VMEM((NBUF, D // 8, 8, BB), jnp.float32),
        pltpu.SemaphoreType.DMA,
        pltpu.SemaphoreType.DMA,
    ],
)
def _embed_kernel(
    xt_hbm, pos_hbm, table_hbm, out_hbm, idx_v, pos_v, bufs, obufs, gsem, wsem
):
    cc = lax.axis_index("c")
    ss = lax.axis_index("s")
    wid = ss * 2 + cc
    tb = wid % NTB
    s0 = (wid // NTB) * SG
    b0 = tb * BB

    pltpu.sync_copy(xt_hbm.at[pl.ds(s0, SG), pl.ds(b0, BB)], idx_v)
    pltpu.sync_copy(pos_hbm.at[pl.ds(s0, SG)], pos_v)

    for b in range(NBUF):
        pltpu.async_copy(table_hbm.at[idx_v.at[b]], bufs.at[b], gsem)

    iota16 = lax.broadcasted_iota(jnp.int32, (16,), 0)
    zeros16 = jnp.zeros((16,), jnp.int32)
    scat = []
    for l in range(D // 16):
        e_vec = iota16 + (l * 16)
        scat.append((lax.shift_right_logical(e_vec, 3), lax.bitwise_and(e_vec, 7)))

    def outer(o, carry):
        for b in range(NBUF):
            blk = o * NBUF + b
            buf = bufs.at[b]
            obuf = obufs.at[b]
            pltpu.make_async_copy(table_hbm.at[idx_v.at[blk]], buf, gsem).wait()
            @pl.when(blk >= NBUF)
            def _():
                pltpu.make_async_copy(obuf, out_hbm.at[s0, :, tb], wsem).wait()

            p0 = pos_v[blk, pl.ds(0, 16)]
            p1 = pos_v[blk, pl.ds(16, 16)]
            p2 = pos_v[blk, pl.ds(32, 16)]
            p3 = pos_v[blk, pl.ds(48, 16)]

            def row(r, pc):
                q0, q1, q2, q3 = pc
                r16 = zeros16 + r
                for l, q in ((0, q0), (1, q1), (2, q2), (3, q3)):
                    v = buf[r, pl.ds(l * 16, 16)] * SCALE + q
                    plsc.store_scatter(obuf, [scat[l][0], scat[l][1], r16], v)
                return pc

            lax.fori_loop(0, BB, row, (p0, p1, p2, p3))

            pltpu.async_copy(obuf, out_hbm.at[s0 + blk, :, tb], wsem)

            @pl.when(blk + NBUF < SG)
            def _():
                pltpu.async_copy(table_hbm.at[idx_v.at[blk + NBUF]], bufs.at[b], gsem)
        return carry

    lax.fori_loop(0, SG // NBUF, outer, 0)

    for b in range(NBUF):
        pltpu.make_async_copy(obufs.at[b], out_hbm.at[s0, :, tb], wsem).wait()


def kernel(x, table):
    xt = jnp.transpose(x.astype(jnp.int32))
    out = _embed_kernel(xt, _POS, table)
    return jnp.reshape(jnp.transpose(out, (2, 4, 0, 1, 3)), (BATCH, SEQ, D))


# speedup vs baseline: 1.3292x; 1.2577x over previous
import functools

import jax
import jax.numpy as jnp
import numpy as np
from jax import lax
from jax.experimental import pallas as pl
from jax.experimental.pallas import tpu as pltpu
from jax.experimental.pallas import tpu_sc as plsc

VOCAB = 1000000
D = 64
BATCH = 1024
SEQ = 200
NW = 32
NTB = 8
BB = BATCH // NTB
NSG = NW // NTB
SG = SEQ // NSG
NBUF = 5
SCALE = 8.0


def _pos_encoding() -> np.ndarray:
    depth = D / 2
    positions = np.arange(SEQ)[:, np.newaxis]
    depths = np.arange(depth)[np.newaxis, :] / depth
    angle_rates = 1 / 10000 ** depths
    angle_rads = positions * angle_rates
    pe = np.concatenate([np.sin(angle_rads), np.cos(angle_rads)], axis=-1)
    return np.asarray(pe, dtype=np.float32)


_POS = _pos_encoding()

_mesh = plsc.VectorSubcoreMesh(core_axis_name="c", subcore_axis_name="s")


@functools.partial(
    pl.kernel,
    mesh=_mesh,
    compiler_params=pltpu.CompilerParams(use_tc_tiling_on_sc=False),
    out_type=jax.ShapeDtypeStruct((BATCH, SEQ // 2, 2 * D), jnp.float32),
    scratch_types=[
        pltpu.VMEM((SG, BB), jnp.int32),
        pltpu.VMEM((SG, D), jnp.float32),
        pltpu.VMEM((NBUF, BB, D), jnp.float32),
        pltpu.VMEM((NBUF, BB, D), jnp.float32),
        pltpu.SemaphoreType.DMA,
        pltpu.SemaphoreType.DMA,
    ],
)
def _embed_kernel(
    xt_hbm, pos_hbm, table_hbm, out_hbm, idx_v, pos_v, bufs, obufs, gsem, wsem
):
    cc = lax.axis_index("c")
    ss = lax.axis_index("s")
    wid = ss * 2 + cc
    tb = wid % NTB
    s0 = (wid // NTB) * SG
    b0 = tb * BB

    pltpu.sync_copy(xt_hbm.at[pl.ds(s0, SG), pl.ds(b0, BB)], idx_v)
    pltpu.sync_copy(pos_hbm.at[pl.ds(s0, SG)], pos_v)

    for b in range(NBUF):
        pltpu.async_copy(table_hbm.at[idx_v.at[b]], bufs.at[b], gsem)

    def outer(o, carry):
        for b in range(NBUF):
            blk = o * NBUF + b
            buf = bufs.at[b]
            obuf = obufs.at[b]
            pltpu.make_async_copy(table_hbm.at[idx_v.at[blk]], buf, gsem).wait()
            @pl.when(blk >= NBUF)
            def _():
                pltpu.make_async_copy(
                    obuf, out_hbm.at[pl.ds(b0, BB), 0, pl.ds(0, D)], wsem
                ).wait()

            p0 = pos_v[blk, pl.ds(0, 16)]
            p1 = pos_v[blk, pl.ds(16, 16)]
            p2 = pos_v[blk, pl.ds(32, 16)]
            p3 = pos_v[blk, pl.ds(48, 16)]

            def row(r, pc):
                q0, q1, q2, q3 = pc
                obuf[r, pl.ds(0, 16)] = buf[r, pl.ds(0, 16)] * SCALE + q0
                obuf[r, pl.ds(16, 16)] = buf[r, pl.ds(16, 16)] * SCALE + q1
                obuf[r, pl.ds(32, 16)] = buf[r, pl.ds(32, 16)] * SCALE + q2
                obuf[r, pl.ds(48, 16)] = buf[r, pl.ds(48, 16)] * SCALE + q3
                return pc

            lax.fori_loop(0, BB, row, (p0, p1, p2, p3))

            s = s0 + blk
            pltpu.async_copy(
                obuf,
                out_hbm.at[pl.ds(b0, BB), s // 2, pl.ds((s % 2) * D, D)],
                wsem,
            )

            @pl.when(blk + NBUF < SG)
            def _():
                pltpu.async_copy(table_hbm.at[idx_v.at[blk + NBUF]], bufs.at[b], gsem)
        return carry

    lax.fori_loop(0, SG // NBUF, outer, 0)

    for b in range(NBUF):
        pltpu.make_async_copy(
            obufs.at[b], out_hbm.at[pl.ds(b0, BB), 0, pl.ds(0, D)], wsem
        ).wait()


def kernel(x, table):
    xt = jnp.transpose(x.astype(jnp.int32))
    out = _embed_kernel(xt, _POS, table)
    return jnp.reshape(out, (BATCH, SEQ, D))
